# Initial kernel scaffold; baseline (speedup 1.0000x reference)
import jax
import jax.numpy as jnp
from jax.experimental import pallas as pl


def kernel(entity_table, attention, w1_0, b1_0, w2_0, b2_0, w1_1, b1_1, w2_1, b2_1, edge_index, userids, itemids_pos, itemids_neg):
    raise NotImplementedError("write your pallas kernel here")


# speedup vs baseline: 2.7278x; 2.7278x over previous
import functools

import jax
import jax.numpy as jnp
from jax import lax
from jax.experimental import pallas as pl
from jax.experimental.pallas import tpu as pltpu
from jax.experimental.pallas import tpu_sc as plsc

N = 10000
E = 320000
D = 128
B = 1024
REG = 1e-05

NC = 2
NS = 16
NW = NC * NS
EPW = E // NW
CHUNK = 80
NCHUNK = EPW // CHUNK
NROWCH = N // CHUNK
BPW = B // NW

_mesh = plsc.VectorSubcoreMesh(core_axis_name="c", subcore_axis_name="s")


def _sc_scatter_body(ego_hbm, src_hbm, dst_hbm, att_hbm, out_hbm,
                     acc, ev, av, rows, gsem):
    c = lax.axis_index("c")
    s = lax.axis_index("s")
    w = s * NC + c
    zlo = (s * NROWCH) // NS
    zhi = ((s + 1) * NROWCH) // NS

    def zero_body(e, _):
        zero = jnp.zeros((16,), jnp.float32)
        for j in range(D // 16):
            rows[e, pl.ds(j * 16, 16)] = zero
        return 0

    lax.fori_loop(0, CHUNK, zero_body, 0)

    def zero_acc_body(k, _):
        row0 = pl.multiple_of(k * CHUNK, 8)
        pltpu.sync_copy(rows, acc.at[pl.ds(row0, CHUNK)])
        return 0

    lax.fori_loop(zlo, zhi, zero_acc_body, 0)
    plsc.subcore_barrier()

    def edge_body(g, _):
        base = pl.multiple_of(w * EPW + g * CHUNK, 8)
        pltpu.sync_copy(src_hbm.at[pl.ds(base, CHUNK)], ev.at[0])
        pltpu.sync_copy(dst_hbm.at[pl.ds(base, CHUNK)], ev.at[1])
        pltpu.sync_copy(att_hbm.at[pl.ds(base * 16, CHUNK * 16)], av)
        pltpu.async_copy(ego_hbm.at[ev.at[0]], rows, gsem).wait()

        def scale_body(e, _):
            ab = av[pl.ds(e * 16, 16)]
            for j in range(D // 16):
                rows[e, pl.ds(j * 16, 16)] = rows[e, pl.ds(j * 16, 16)] * ab
            return 0

        lax.fori_loop(0, CHUNK, scale_body, 0)
        pltpu.sync_copy(rows, acc.at[ev.at[1]], add=True)
        return 0

    lax.fori_loop(0, NCHUNK, edge_body, 0)
    plsc.subcore_barrier()

    def copyout_body(k, _):
        row0 = pl.multiple_of(k * CHUNK, 8)
        pltpu.sync_copy(acc.at[pl.ds(row0, CHUNK)], rows)
        pltpu.sync_copy(rows, out_hbm.at[c, pl.ds(row0, CHUNK)])
        return 0

    lax.fori_loop(zlo, zhi, copyout_body, 0)


_sc_scatter = pl.kernel(
    _sc_scatter_body,
    out_type=jax.ShapeDtypeStruct((NC, N, D), jnp.float32),
    mesh=_mesh,
    scratch_types=[
        pltpu.VMEM_SHARED((N, D), jnp.float32),
        pltpu.VMEM((2, CHUNK), jnp.int32),
        pltpu.VMEM((CHUNK * 16,), jnp.float32),
        pltpu.VMEM((CHUNK, D), jnp.float32),
        pltpu.SemaphoreType.DMA,
    ],
)


def _sc_gather_body(t0, t1, t2, ids_hbm, out_hbm, idxv, rowsv, sem):
    c = lax.axis_index("c")
    s = lax.axis_index("s")
    w = s * NC + c
    base = pl.multiple_of(w * BPW, 8)
    for q in range(3):
        pltpu.sync_copy(ids_hbm.at[pl.ds(q * B + base, BPW)], idxv.at[0])
        for t, tab in enumerate((t0, t1, t2)):
            pltpu.async_copy(tab.at[idxv.at[0]], rowsv, sem).wait()
            pltpu.sync_copy(rowsv, out_hbm.at[t, q, pl.ds(base, BPW)])


_sc_gather = pl.kernel(
    _sc_gather_body,
    out_type=jax.ShapeDtypeStruct((3, 3, B, D), jnp.float32),
    mesh=_mesh,
    scratch_types=[
        pltpu.VMEM((1, BPW), jnp.int32),
        pltpu.VMEM((BPW, D), jnp.float32),
        pltpu.SemaphoreType.DMA,
    ],
)


def _tc_layer_body(ego_ref, p_ref, w1_ref, b1_ref, w2_ref, b2_ref,
                   oego_ref, onorm_ref):
    ego = ego_ref[...]
    agg = p_ref[0] + p_ref[1]
    h1 = jnp.dot(ego + agg, w1_ref[...],
                 preferred_element_type=jnp.float32) + b1_ref[...]
    h1 = jnp.where(h1 >= 0, h1, 0.01 * h1)
    h2 = jnp.dot(ego * agg, w2_ref[...],
                 preferred_element_type=jnp.float32) + b2_ref[...]
    h2 = jnp.where(h2 >= 0, h2, 0.01 * h2)
    newego = h1 + h2
    oego_ref[...] = newego
    nrm = jnp.sqrt(jnp.sum(newego * newego, axis=1, keepdims=True)) + 1e-12
    onorm_ref[...] = newego / nrm


_TC_R = 1000


def _tc_layer(ego, parts, w1, b1, w2, b2):
    grid = (N // _TC_R,)
    return pl.pallas_call(
        _tc_layer_body,
        grid=grid,
        in_specs=[
            pl.BlockSpec((_TC_R, D), lambda i: (i, 0)),
            pl.BlockSpec((NC, _TC_R, D), lambda i: (0, i, 0)),
            pl.BlockSpec((D, D), lambda i: (0, 0)),
            pl.BlockSpec((1, D), lambda i: (0, 0)),
            pl.BlockSpec((D, D), lambda i: (0, 0)),
            pl.BlockSpec((1, D), lambda i: (0, 0)),
        ],
        out_specs=[
            pl.BlockSpec((_TC_R, D), lambda i: (i, 0)),
            pl.BlockSpec((_TC_R, D), lambda i: (i, 0)),
        ],
        out_shape=[
            jax.ShapeDtypeStruct((N, D), jnp.float32),
            jax.ShapeDtypeStruct((N, D), jnp.float32),
        ],
    )(ego, parts, w1, b1.reshape(1, D), w2, b2.reshape(1, D))


def _tc_loss_body(g_ref, base_ref, reg_ref):
    sp = jnp.zeros((B, 1), jnp.float32)
    sn = jnp.zeros((B, 1), jnp.float32)
    l2 = jnp.float32(0.0)
    for t in range(3):
        u = g_ref[t, 0]
        p = g_ref[t, 1]
        n = g_ref[t, 2]
        sp = sp + jnp.sum(u * p, axis=1, keepdims=True)
        sn = sn + jnp.sum(u * n, axis=1, keepdims=True)
        l2 = l2 + jnp.sum(u * u) + jnp.sum(p * p) + jnp.sum(n * n)
    x = -(sp - sn)
    softplus = jnp.maximum(x, 0.0) + jnp.log1p(jnp.exp(-jnp.abs(x)))
    base_ref[...] = jnp.sum(softplus).reshape(1, 1)
    reg_ref[...] = (jnp.float32(REG) * 0.5 * l2).reshape(1, 1)


def _tc_loss(gathered):
    return pl.pallas_call(
        _tc_loss_body,
        out_shape=[
            jax.ShapeDtypeStruct((1, 1), jnp.float32),
            jax.ShapeDtypeStruct((1, 1), jnp.float32),
        ],
    )(gathered)


def kernel(entity_table, attention, w1_0, b1_0, w2_0, b2_0, w1_1, b1_1,
           w2_1, b2_1, edge_index, userids, itemids_pos, itemids_neg):
    src = edge_index[0]
    dst = edge_index[1]
    att = jnp.broadcast_to(attention, (E, 16)).reshape(E * 16)

    parts0 = _sc_scatter(entity_table, src, dst, att)
    ego1, norm1 = _tc_layer(entity_table, parts0, w1_0, b1_0, w2_0, b2_0)
    parts1 = _sc_scatter(ego1, src, dst, att)
    _, norm2 = _tc_layer(ego1, parts1, w1_1, b1_1, w2_1, b2_1)

    ids = jnp.concatenate([userids, itemids_pos, itemids_neg], axis=0)
    gathered = _sc_gather(entity_table, norm1, norm2, ids)
    base, reg = _tc_loss(gathered)
    return (base.reshape(()), reg.reshape(()))
